# baseline (speedup 1.0000x reference)
import functools

import jax
import jax.numpy as jnp
from jax import lax
from jax.experimental import pallas as pl
from jax.experimental.pallas import tpu as pltpu
from jax.experimental.pallas import tpu_sc as plsc

VOCAB = 1_000_000
DIM = 64

NC = 2
NS = 16
NW = NC * NS

CH = 128


def _emb_body(seq, n_chunks, w2, xT, outT, ia_all,
              idx2_0, idx2_1, idx2_2, idx2_3,
              cb_0, cb_1, cb_2, cb_3,
              rows0, rows1, rows2, rows3, tr0, tr1,
              gsem0, gsem1, gsem2, gsem3, osem0, osem1):
    wid = lax.axis_index("s") * NC + lax.axis_index("c")
    cpw = n_chunks // NW
    c_base = wid * cpw
    srows = (seq + 7) // 8 * 8
    iota = lax.iota(jnp.int32, 16)

    for j in range(cpw):
        pltpu.sync_copy(
            xT.at[:, pl.ds((c_base + j) * CH, CH)],
            ia_all.at[pl.ds(j * srows, seq), :],
        )

    def prep(row, idx2b, cbb, rowsb, gsemb):
        for g in range(CH // 16):
            vi = ia_all[row, pl.ds(g * 16, 16)]
            idx2b[pl.ds(g * 16, 16)] = lax.shift_right_logical(vi, 1)
            cbb[pl.ds(g * 16, 16)] = lax.shift_left(
                lax.bitwise_and(vi, 1), 6)
        return pltpu.async_copy(w2.at[idx2b], rowsb, gsemb)

    def transpose(cbb, rowsb, trb):
        @pl.loop(0, CH // 16, unroll=4)
        def _g(g):
            g16 = g * 16
            rid = g16 + iota
            cb = cbb[pl.ds(g16, 16)]
            for d0 in range(0, DIM, 16):
                cbd = cb + d0
                for k in range(16):
                    perm = lax.bitwise_and(iota + k, 15)
                    vals = plsc.load_gather(rowsb, [rid, cbd + perm])
                    plsc.store_scatter(trb, [perm + d0, rid], vals)

    def fire_out(s, cglob, trb, osemb):
        return pltpu.async_copy(
            trb, outT.at[s, :, pl.ds(cglob * CH, CH)], osemb)

    def drain_out(trb, osemb):
        pltpu.make_async_copy(
            trb, outT.at[0, :, pl.ds(0, CH)], osemb).wait()

    def task_coords(t):
        j = t // seq
        s = t - j * seq
        return j, s

    idx2s = [idx2_0, idx2_1, idx2_2, idx2_3]
    cbbs = [cb_0, cb_1, cb_2, cb_3]
    rowss = [rows0, rows1, rows2, rows3]
    trs = [tr0, tr1]
    gsems = [gsem0, gsem1, gsem2, gsem3]
    osems = [osem0, osem1]

    @pl.loop(0, (cpw * seq) // 4)
    def _quad(i):
        staged = []
        for k in range(4):
            t = 4 * i + k
            j, s = task_coords(t)
            d = prep(j * srows + s, idx2s[k], cbbs[k], rowss[k], gsems[k])
            staged.append((s, c_base + j, d))
        for k in range(4):
            s, cglob, d = staged[k]
            d.wait()
            if k < 2:
                @pl.when(i > 0)
                def _():
                    drain_out(trs[k % 2], osems[k % 2])
            else:
                drain_out(trs[k % 2], osems[k % 2])
            transpose(cbbs[k], rowss[k], trs[k % 2])
            fire_out(s, cglob, trs[k % 2], osems[k % 2])

    drain_out(tr0, osem0)
    drain_out(tr1, osem1)


@functools.partial(jax.jit, static_argnames=("seq", "n_chunks"))
def _emb(w2, xT, seq, n_chunks):
    mesh = plsc.VectorSubcoreMesh(
        core_axis_name="c", subcore_axis_name="s", num_cores=NC, num_subcores=NS
    )
    batch = n_chunks * CH
    srows = (seq + 7) // 8 * 8
    cpw = n_chunks // NW
    return pl.kernel(
        functools.partial(_emb_body, seq, n_chunks),
        out_type=jax.ShapeDtypeStruct((seq, DIM, batch), jnp.float32),
        mesh=mesh,
        scratch_types=[
            pltpu.VMEM((cpw * srows, CH), jnp.int32),
            pltpu.VMEM((CH,), jnp.int32),
            pltpu.VMEM((CH,), jnp.int32),
            pltpu.VMEM((CH,), jnp.int32),
            pltpu.VMEM((CH,), jnp.int32),
            pltpu.VMEM((CH,), jnp.int32),
            pltpu.VMEM((CH,), jnp.int32),
            pltpu.VMEM((CH,), jnp.int32),
            pltpu.VMEM((CH,), jnp.int32),
            pltpu.VMEM((CH, 128), jnp.float32),
            pltpu.VMEM((CH, 128), jnp.float32),
            pltpu.VMEM((CH, 128), jnp.float32),
            pltpu.VMEM((CH, 128), jnp.float32),
            pltpu.VMEM((DIM, CH), jnp.float32),
            pltpu.VMEM((DIM, CH), jnp.float32),
            pltpu.SemaphoreType.DMA,
            pltpu.SemaphoreType.DMA,
            pltpu.SemaphoreType.DMA,
            pltpu.SemaphoreType.DMA,
            pltpu.SemaphoreType.DMA,
            pltpu.SemaphoreType.DMA,
        ],
        compiler_params=pltpu.CompilerParams(
            use_tc_tiling_on_sc=True, needs_layout_passes=False),
    )(w2, xT)


def kernel(x, weight):
    b, s = x.shape
    assert b % CH == 0 and (b // CH) % NW == 0 and s % 2 == 0
    w2 = jnp.reshape(weight, (VOCAB // 2, 2 * DIM))
    xT = x.T.astype(jnp.int32)
    outT = _emb(w2, xT, s, b // CH)
    return outT.transpose(2, 0, 1)


# speedup vs baseline: 1.2495x; 1.2495x over previous
import functools

import jax
import jax.numpy as jnp
from jax import lax
from jax.experimental import pallas as pl
from jax.experimental.pallas import tpu as pltpu
from jax.experimental.pallas import tpu_sc as plsc

VOCAB = 1_000_000
DIM = 64

NC = 2
NS = 16
NW = NC * NS

CH = 128


def _emb_body(seq, n_chunks, w2, xT, outT, ia_all,
              idx2_0, idx2_1, idx2_2, idx2_3,
              cb_0, cb_1, cb_2, cb_3,
              rows0, rows1, rows2, rows3, tr0, tr1,
              gsem0, gsem1, gsem2, gsem3, osem0, osem1):
    wid = lax.axis_index("s") * NC + lax.axis_index("c")
    cpw = n_chunks // NW
    c_base = wid * cpw
    srows = (seq + 7) // 8 * 8
    iota = lax.iota(jnp.int32, 16)

    for j in range(cpw):
        pltpu.sync_copy(
            xT.at[:, pl.ds((c_base + j) * CH, CH)],
            ia_all.at[pl.ds(j * srows, seq), :],
        )

    def prep(row, idx2b, cbb, rowsb, gsemb):
        for g in range(CH // 16):
            vi = ia_all[row, pl.ds(g * 16, 16)]
            idx2b[pl.ds(g * 16, 16)] = lax.shift_right_logical(vi, 1)
            cbb[pl.ds(g * 16, 16)] = lax.shift_left(
                lax.bitwise_and(vi, 1), 6)
        return pltpu.async_copy(w2.at[idx2b], rowsb, gsemb)

    def transpose(cbb, rowsb, trb):
        @pl.loop(0, CH // 16, unroll=2)
        def _g(g):
            g16 = g * 16
            rid = g16 + iota
            cb = cbb[pl.ds(g16, 16)]
            for d0 in range(0, DIM, 16):
                cbd = cb + d0
                for k in range(16):
                    perm = lax.bitwise_and(iota + k, 15)
                    vals = plsc.load_gather(rowsb, [rid, cbd + perm])
                    plsc.store_scatter(trb, [perm + d0, rid], vals)

    def fire_out(s, cglob, trb, osemb):
        return pltpu.async_copy(
            trb, outT.at[s, :, pl.ds(cglob * CH, CH)], osemb)

    def drain_out(trb, osemb):
        pltpu.make_async_copy(
            trb, outT.at[0, :, pl.ds(0, CH)], osemb).wait()

    def task_coords(t):
        j = t // seq
        s = t - j * seq
        return j, s

    idx2s = [idx2_0, idx2_1, idx2_2, idx2_3]
    cbbs = [cb_0, cb_1, cb_2, cb_3]
    rowss = [rows0, rows1, rows2, rows3]
    trs = [tr0, tr1]
    gsems = [gsem0, gsem1, gsem2, gsem3]
    osems = [osem0, osem1]

    @pl.loop(0, (cpw * seq) // 4)
    def _quad(i):
        staged = []
        for k in range(4):
            t = 4 * i + k
            j, s = task_coords(t)
            d = prep(j * srows + s, idx2s[k], cbbs[k], rowss[k], gsems[k])
            staged.append((s, c_base + j, d))
        for k in range(4):
            s, cglob, d = staged[k]
            d.wait()
            if k < 2:
                @pl.when(i > 0)
                def _():
                    drain_out(trs[k % 2], osems[k % 2])
            else:
                drain_out(trs[k % 2], osems[k % 2])
            transpose(cbbs[k], rowss[k], trs[k % 2])
            fire_out(s, cglob, trs[k % 2], osems[k % 2])

    drain_out(tr0, osem0)
    drain_out(tr1, osem1)


@functools.partial(jax.jit, static_argnames=("seq", "n_chunks"))
def _emb(w2, xT, seq, n_chunks):
    mesh = plsc.VectorSubcoreMesh(
        core_axis_name="c", subcore_axis_name="s", num_cores=NC, num_subcores=NS
    )
    batch = n_chunks * CH
    srows = (seq + 7) // 8 * 8
    cpw = n_chunks // NW
    return pl.kernel(
        functools.partial(_emb_body, seq, n_chunks),
        out_type=jax.ShapeDtypeStruct((seq, DIM, batch), jnp.float32),
        mesh=mesh,
        scratch_types=[
            pltpu.VMEM((cpw * srows, CH), jnp.int32),
            pltpu.VMEM((CH,), jnp.int32),
            pltpu.VMEM((CH,), jnp.int32),
            pltpu.VMEM((CH,), jnp.int32),
            pltpu.VMEM((CH,), jnp.int32),
            pltpu.VMEM((CH,), jnp.int32),
            pltpu.VMEM((CH,), jnp.int32),
            pltpu.VMEM((CH,), jnp.int32),
            pltpu.VMEM((CH,), jnp.int32),
            pltpu.VMEM((CH, 128), jnp.float32),
            pltpu.VMEM((CH, 128), jnp.float32),
            pltpu.VMEM((CH, 128), jnp.float32),
            pltpu.VMEM((CH, 128), jnp.float32),
            pltpu.VMEM((DIM, CH), jnp.float32),
            pltpu.VMEM((DIM, CH), jnp.float32),
            pltpu.SemaphoreType.DMA,
            pltpu.SemaphoreType.DMA,
            pltpu.SemaphoreType.DMA,
            pltpu.SemaphoreType.DMA,
            pltpu.SemaphoreType.DMA,
            pltpu.SemaphoreType.DMA,
        ],
        compiler_params=pltpu.CompilerParams(
            use_tc_tiling_on_sc=True, needs_layout_passes=False),
    )(w2, xT)


def kernel(x, weight):
    b, s = x.shape
    assert b % CH == 0 and (b // CH) % NW == 0 and s % 2 == 0
    w2 = jnp.reshape(weight, (VOCAB // 2, 2 * DIM))
    xT = x.T.astype(jnp.int32)
    outT = _emb(w2, xT, s, b // CH)
    return outT.transpose(2, 0, 1)
